# Initial kernel scaffold; baseline (speedup 1.0000x reference)
import jax
import jax.numpy as jnp
from jax.experimental import pallas as pl


def kernel(fine_h_A, fine_x_A, coarse_h_A, pool_h_A, pool_x_A, og_pool_x_A, edge_feat_A, src_A, dst_A, fine_h_B, fine_x_B, coarse_h_B, pool_h_B, pool_x_B, og_pool_x_B, edge_feat_B, src_B, dst_B, params):
    raise NotImplementedError("write your pallas kernel here")


# speedup vs baseline: 3.3307x; 3.3307x over previous
import functools

import jax
import jax.numpy as jnp
import numpy as np
from jax import lax
from jax.experimental import pallas as pl
from jax.experimental.pallas import tpu as pltpu
from jax.experimental.pallas import tpu_sc as plsc

_DH = 128
_NF = 10000
_NC = 1000
_E = 320000
_NEG = 0.01
_SKIP_H = 0.5
_X_INIT = 0.25

_B = 2000
_NW = 32
_EPW = _E // _NW
_R = 80
_NCH = _EPW // _R


def _lrelu(x):
    return jnp.where(x > 0, x, _NEG * x)


def _ln(x, g, b):
    m = jnp.mean(x, axis=-1, keepdims=True)
    v = jnp.mean((x - m) * (x - m), axis=-1, keepdims=True)
    return (x - m) / jnp.sqrt(v + 1e-5) * g + b


def _cumsum_rows(x, n):
    s = 1
    while s < n:
        shifted = jnp.concatenate([jnp.zeros((s, 1), x.dtype), x[: n - s, :]], axis=0)
        x = x + shifted
        s *= 2
    return x


_MASKHI = -65536


def _pre_body(fh_ref, fx_ref, ch_ref, wf_ref, wc_ref, eb_ref, t_ref, chp_ref):
    i32 = jnp.int32
    f = jnp.dot(fh_ref[:], wf_ref[:], preferred_element_type=jnp.float32)
    ai = lax.bitcast_convert_type(f[:, 0:64], i32)
    bi = lax.bitcast_convert_type(f[:, 64:128], i32)
    hi = (ai + 0x8000) & _MASKHI
    lo = lax.shift_right_logical((bi + 0x8000) & _MASKHI, 16)
    t_ref[:, 0:64] = lax.bitcast_convert_type(hi | lo, jnp.float32)
    t_ref[:, 64:128] = jnp.concatenate(
        [fx_ref[:], jnp.zeros((_NF, 61), jnp.float32)], axis=1)
    chp_ref[:] = jnp.dot(ch_ref[:], wc_ref[:], preferred_element_type=jnp.float32) + eb_ref[:]


def _precompute(fine_h, fine_x, coarse_h, wf, wc, eb1r):
    return pl.pallas_call(
        _pre_body,
        out_shape=(
            jax.ShapeDtypeStruct((_NF, 128), jnp.float32),
            jax.ShapeDtypeStruct((_NC, 128), jnp.float32),
        ),
    )(fine_h, fine_x, coarse_h, wf, wc, eb1r)


def _sc_gather(t_table, src32):
    mesh = plsc.VectorSubcoreMesh(core_axis_name="c", subcore_axis_name="s")

    @functools.partial(
        pl.kernel,
        mesh=mesh,
        out_type=jax.ShapeDtypeStruct((_E, 128), jnp.float32),
        scratch_types=[
            pltpu.VMEM((_R,), jnp.int32),
            pltpu.VMEM((_R, 128), jnp.float32),
            pltpu.SemaphoreType.DMA,
        ],
    )
    def gat(src_hbm, t_hbm, g_hbm, idx_v, rows_v, sem):
        wid = lax.axis_index("s") * 2 + lax.axis_index("c")
        base = wid * _EPW

        def chunk(c, carry):
            off = base + c * _R
            pltpu.sync_copy(src_hbm.at[pl.ds(off, _R)], idx_v)
            pltpu.async_copy(t_hbm.at[idx_v], rows_v, sem).wait()
            pltpu.sync_copy(rows_v, g_hbm.at[pl.ds(off, _R)])
            return carry

        lax.fori_loop(0, _NCH, chunk, 0)

    return gat(src32, t_table)


def _edge_body(g_ref, ef_ref, dst_ref, ch_ref, cx_ref, wfr_ref, ew2_ref,
               cw1_ref, pv_ref, accm_ref, acca_ref,
               exph_ref, expx_ref, carm_ref, cara_ref, cdst_ref):
    pid = pl.program_id(0)
    nblk = pl.num_programs(0)
    f32 = jnp.float32

    d = dst_ref[:]

    @pl.when(pid == 0)
    def _init():
        accm_ref[:] = jnp.zeros((_NC, 128), f32)
        acca_ref[:] = jnp.zeros((_NC, 16), f32)
        carm_ref[:] = jnp.zeros((1, 128), f32)
        cara_ref[:] = jnp.zeros((1, 16), f32)
        cdst_ref[0] = jnp.min(d)

    prev_dst = cdst_ref[0]
    d_shift = jnp.concatenate([jnp.reshape(prev_dst, (1, 1)), d[: _B - 1, :]], axis=0)
    is_b = (d != d_shift).astype(jnp.int32)
    seg = _cumsum_rows(is_b, _B)
    q = jnp.max(seg)

    exph_ref[:] = jnp.zeros((_B, 128), f32)
    expx_ref[:] = jnp.zeros((_B, 16), f32)

    def exp_body(k, carry):
        mask = seg == k
        dk = jnp.maximum(jnp.max(jnp.where(mask, d, -1)), 0)
        mf = mask.astype(f32)
        exph_ref[:] += mf * ch_ref[pl.ds(dk, 1), :]
        expx_ref[:] += mf * cx_ref[pl.ds(dk, 1), :]
        return carry

    lax.fori_loop(0, q + 1, exp_body, 0)

    x_rel3 = g_ref[:, 64:67] - expx_ref[:, 0:3]
    mag = jnp.sum(x_rel3 * x_rel3, axis=1, keepdims=True)
    inv_s = pv_ref[8:9, 0:16]
    rbf16 = jnp.exp(-mag * inv_s)
    feats = jnp.concatenate([ef_ref[:], rbf16], axis=1)

    gi = lax.bitcast_convert_type(g_ref[:, 0:64], jnp.int32)
    fa = lax.bitcast_convert_type(gi & _MASKHI, f32)
    fb = lax.bitcast_convert_type(lax.shift_left(gi, 16), f32)
    fg = jnp.concatenate([fa, fb], axis=1)

    pre1 = (fg + exph_ref[:]
            + jnp.dot(feats, wfr_ref[:], preferred_element_type=f32))
    h1 = _lrelu(_ln(pre1, pv_ref[0:1, :], pv_ref[1:2, :]))
    msg = _ln(jnp.dot(h1, ew2_ref[:], preferred_element_type=f32) + pv_ref[2:3, :],
              pv_ref[3:4, :], pv_ref[4:5, :])
    t = _lrelu(jnp.dot(msg, cw1_ref[:], preferred_element_type=f32) + pv_ref[5:6, :])
    coef = jnp.sum(t * pv_ref[6:7, :], axis=1, keepdims=True) + pv_ref[7:8, 0:1]
    upd3 = x_rel3 * coef
    aux = jnp.concatenate(
        [upd3, jnp.ones((_B, 1), f32), jnp.zeros((_B, 12), f32)], axis=1)

    def fl_body(k, carry):
        mask = seg == k
        mf = mask.astype(f32)
        summ = jnp.sum(mf * msg, axis=0, keepdims=True)
        suma = jnp.sum(mf * aux, axis=0, keepdims=True)
        dk = jnp.maximum(jnp.max(jnp.where(mask, d, -1)), 0)
        is0 = k == 0
        valm = jnp.where(is0, carm_ref[:] + summ, summ)
        vala = jnp.where(is0, cara_ref[:] + suma, suma)
        fdst = jnp.where(is0, cdst_ref[0], dk)
        do_flush = (k < q).astype(f32)
        accm_ref[pl.ds(fdst, 1), :] += do_flush * valm
        acca_ref[pl.ds(fdst, 1), :] += do_flush * vala

        @pl.when(k == q)
        def _set_carry():
            carm_ref[:] = valm
            cara_ref[:] = vala

        return carry

    lax.fori_loop(0, q + 1, fl_body, 0)
    cdst_ref[0] = jnp.max(d)

    @pl.when(pid == nblk - 1)
    def _final_flush():
        fd = cdst_ref[0]
        accm_ref[pl.ds(fd, 1), :] += carm_ref[:]
        acca_ref[pl.ds(fd, 1), :] += cara_ref[:]


def _edge(g, ef, dst2, ch, cx, wfr, ew2, cw1, pvec):
    nblk = _E // _B
    blk = lambda shape: pl.BlockSpec(shape, lambda i: (i, 0))
    res = lambda shape: pl.BlockSpec(shape, lambda i: (0, 0))
    return pl.pallas_call(
        _edge_body,
        grid=(nblk,),
        in_specs=[
            blk((_B, 128)), blk((_B, 16)), blk((_B, 1)),
            res((_NC, 128)), res((_NC, 16)), res((32, 128)),
            res((128, 128)), res((128, 128)), res((16, 128)),
        ],
        out_specs=(res((_NC, 128)), res((_NC, 16))),
        out_shape=(
            jax.ShapeDtypeStruct((_NC, 128), jnp.float32),
            jax.ShapeDtypeStruct((_NC, 16), jnp.float32),
        ),
        scratch_shapes=[
            pltpu.VMEM((_B, 128), jnp.float32),
            pltpu.VMEM((_B, 16), jnp.float32),
            pltpu.VMEM((1, 128), jnp.float32),
            pltpu.VMEM((1, 16), jnp.float32),
            pltpu.SMEM((1,), jnp.int32),
        ],
    )(g, ef, dst2, ch, cx, wfr, ew2, cw1, pvec)


def _node_body(accm_ref, acca_ref, ph_ref, px_ref, og_ref, w1a_ref, w1b_ref,
               w2_ref, nv_ref, xf_ref, hf_ref):
    f32 = jnp.float32
    cnt = jnp.maximum(acca_ref[:, 3:4], 1.0)
    aggr = accm_ref[:] / cnt
    x_upd = acca_ref[:, 0:3] / cnt
    xf_ref[:] = _X_INIT * og_ref[:, 0:3] + (1.0 - _X_INIT) * px_ref[:, 0:3] + x_upd
    h = (jnp.dot(ph_ref[:], w1a_ref[:], preferred_element_type=f32)
         + jnp.dot(aggr, w1b_ref[:], preferred_element_type=f32) + nv_ref[0:1, :])
    h = _lrelu(_ln(h, nv_ref[1:2, :], nv_ref[2:3, :]))
    nout = _ln(jnp.dot(h, w2_ref[:], preferred_element_type=f32) + nv_ref[3:4, :],
               nv_ref[4:5, :], nv_ref[5:6, :])
    hf_ref[:] = _SKIP_H * nout + (1.0 - _SKIP_H) * ph_ref[:]


def _node(accm, acca, pool_h, px16, og16, w1a, w1b, w2, nvec):
    return pl.pallas_call(
        _node_body,
        out_shape=(
            jax.ShapeDtypeStruct((_NC, 3), jnp.float32),
            jax.ShapeDtypeStruct((_NC, 128), jnp.float32),
        ),
    )(accm, acca, pool_h, px16, og16, w1a, w1b, w2, nvec)


def _one_graph(p_pack, fine_h, fine_x, coarse_h, pool_h, pool_x, og_pool_x,
               e_feat, src, dst):
    wf, wc, eb1r, wfr, ew2, cw1, pvec, w1a, w1b, w2, nvec = p_pack
    px16 = jnp.pad(pool_x, ((0, 0), (0, 13)))
    og16 = jnp.pad(og_pool_x, ((0, 0), (0, 13)))
    t_table, ch = _precompute(fine_h, fine_x, coarse_h, wf, wc, eb1r)
    g = _sc_gather(t_table, src.astype(jnp.int32))
    dst2 = dst.astype(jnp.int32).reshape(_E, 1)
    accm, acca = _edge(g, e_feat, dst2, ch, px16, wfr, ew2, cw1, pvec)
    return _node(accm, acca, pool_h, px16, og16, w1a, w1b, w2, nvec)


def kernel(fine_h_A, fine_x_A, coarse_h_A, pool_h_A, pool_x_A, og_pool_x_A,
           edge_feat_A, src_A, dst_A, fine_h_B, fine_x_B, coarse_h_B, pool_h_B,
           pool_x_B, og_pool_x_B, edge_feat_B, src_B, dst_B, params):
    p = params
    wf = p['eW1'][0:128]
    wc = p['eW1'][128:256]
    wfr = jnp.concatenate([p['eW1'][256:287], jnp.zeros((1, 128), jnp.float32)], axis=0)
    eb1r = p['eb1'].reshape(1, 128)
    inv_s = np.zeros((128,), np.float32)
    inv_s[:15] = [1.0 / (1.5 ** i) for i in range(15)]
    zrow = jnp.zeros((128,), jnp.float32)
    pvec = jnp.stack([
        p['eg1'], p['ebn1'], p['eb2'], p['eg2'], p['ebn2'], p['cb1'],
        p['cW2'][:, 0], jnp.broadcast_to(p['cb2'], (128,)),
        jnp.asarray(inv_s),
        zrow, zrow, zrow, zrow, zrow, zrow, zrow,
    ])
    nvec = jnp.stack([
        p['nb1'], p['ng1'], p['nbn1'], p['nb2'], p['ng2'], p['nbn2'], zrow, zrow,
    ])
    p_pack = (wf, wc, eb1r, wfr, p['eW2'], p['cW1'], pvec,
              p['nW1'][0:128], p['nW1'][128:256], p['nW2'], nvec)
    xa, ha = _one_graph(p_pack, fine_h_A, fine_x_A, coarse_h_A, pool_h_A,
                        pool_x_A, og_pool_x_A, edge_feat_A, src_A, dst_A)
    xb, hb = _one_graph(p_pack, fine_h_B, fine_x_B, coarse_h_B, pool_h_B,
                        pool_x_B, og_pool_x_B, edge_feat_B, src_B, dst_B)
    return (xa, ha, xb, hb)
